# baseline (speedup 1.0000x reference)
import jax
import jax.numpy as jnp
from jax.experimental import pallas as pl
from jax.experimental.pallas import tpu as pltpu

B, S, D = 4, 8192, 4096


def _body(x_hbm, o_ref, sem):
    pltpu.make_async_copy(x_hbm.at[:, 0:1, :], o_ref, sem).start()
    pltpu.make_async_copy(x_hbm.at[:, 0:1, :], o_ref, sem).wait()


def kernel(hidden_state):
    return pl.pallas_call(
        _body,
        grid=(1,),
        in_specs=[pl.BlockSpec(memory_space=pl.ANY)],
        out_specs=pl.BlockSpec(memory_space=pl.ANY),
        out_shape=jax.ShapeDtypeStruct((B, 1, D), jnp.float32),
        scratch_shapes=[pltpu.SemaphoreType.DMA],
    )(hidden_state)


# speedup vs baseline: 2.0450x; 2.0450x over previous
import jax
import jax.numpy as jnp
from jax.experimental import pallas as pl
from jax.experimental.pallas import tpu as pltpu

B, S, D = 4, 8192, 4096


def _body(x_hbm, o_ref, sem):
    copy = pltpu.make_async_copy(x_hbm.at[:, 0:1, :], o_ref, sem)
    copy.start()
    copy.wait()


def kernel(hidden_state):
    return pl.pallas_call(
        _body,
        in_specs=[pl.BlockSpec(memory_space=pl.ANY)],
        out_shape=jax.ShapeDtypeStruct((B, 1, D), jnp.float32),
        scratch_shapes=[pltpu.SemaphoreType.DMA],
    )(hidden_state)
